# baseline (speedup 1.0000x reference)
import functools

import jax
import jax.numpy as jnp
from jax import lax
from jax.experimental import pallas as pl
from jax.experimental.pallas import tpu as pltpu
from jax.experimental.pallas import tpu_sc as plsc

EMBED_DIM = 32
CHUNK = 512
VBLK = 512


@functools.lru_cache(maxsize=None)
def _make_reformat(V: int, D: int):
  info = plsc.get_sparse_core_info()
  nc, ns = info.num_cores, info.num_subcores
  nw = nc * ns
  n_full = V // VBLK
  tail = V - n_full * VBLK
  per_w = n_full // nw
  n_extra = n_full - per_w * nw
  rows_blk = VBLK * D // 128
  mesh = plsc.VectorSubcoreMesh(core_axis_name="c", subcore_axis_name="s")
  assert per_w % 2 == 1 and per_w >= 3

  @functools.partial(
      pl.kernel,
      mesh=mesh,
      out_type=jax.ShapeDtypeStruct((V * D // 128, 128), jnp.float32),
      compiler_params=pltpu.CompilerParams(needs_layout_passes=False),
      scratch_types=[
          pltpu.VMEM((D, VBLK), jnp.float32),
          pltpu.VMEM((D, VBLK), jnp.float32),
          pltpu.VMEM((rows_blk, 128), jnp.float32),
          pltpu.VMEM((rows_blk, 128), jnp.float32),
          pltpu.SemaphoreType.DMA,
          pltpu.SemaphoreType.DMA,
          pltpu.SemaphoreType.DMA,
      ],
  )
  def fmt_kernel(tt_hbm, tail_hbm, lin_hbm, bt0, bt1, bo0, bo1,
                 isem, osem0, osem1):
    wid = lax.axis_index("s") * nc + lax.axis_index("c")
    lane = lax.iota(jnp.int32, 16)
    bts = (bt0, bt1)
    bos = (bo0, bo1)
    osems = (osem0, osem1)

    def bid_of(k):
      return wid * per_w + k

    def in_start(k, p):
      v0 = pl.multiple_of(bid_of(k) * VBLK, VBLK)
      pltpu.async_copy(tt_hbm.at[:, pl.ds(v0, VBLK)], bts[p], isem)

    def in_wait(k, p):
      v0 = pl.multiple_of(bid_of(k) * VBLK, VBLK)
      pltpu.make_async_copy(tt_hbm.at[:, pl.ds(v0, VBLK)], bts[p],
                            isem).wait()

    def transpose(p):
      @plsc.parallel_loop(0, VBLK // 16, 1, unroll=4)
      def _tp(t):
        rr = t * 16 + lane
        q = lax.shift_right_logical(rr, 2)
        r32 = lax.shift_left(lax.bitwise_and(rr, 3), 5)
        for d in range(D):
          plsc.store_scatter(bos[p], [q, r32 + d],
                             bts[p][d, pl.ds(t * 16, 16)])

    def out_start(k, p):
      row0 = pl.multiple_of(bid_of(k) * rows_blk, rows_blk)
      pltpu.async_copy(bos[p], lin_hbm.at[pl.ds(row0, rows_blk)], osems[p])

    def out_wait(k, p):
      row0 = pl.multiple_of(bid_of(k) * rows_blk, rows_blk)
      pltpu.make_async_copy(bos[p], lin_hbm.at[pl.ds(row0, rows_blk)],
                            osems[p]).wait()

    def step(k, p, last, first=False):
      in_wait(k, p)
      if not last:
        in_start(k + 1, 1 - p)
      if not first:
        out_wait(k - 2, p)
      transpose(p)
      out_start(k, p)

    in_start(0, 0)
    step(0, 0, False, first=True)
    step(1, 1, False, first=True)

    def body(k2, carry):
      k = 2 * k2
      step(k, 0, False)
      step(k + 1, 1, False)
      return carry

    lax.fori_loop(1, (per_w - 1) // 2, body, 0)
    step(per_w - 1, 0, True)
    out_wait(per_w - 2, 1)
    out_wait(per_w - 1, 0)

    if n_extra:
      @pl.when(wid < n_extra)
      def _():
        bid = n_full - n_extra + wid
        v0 = pl.multiple_of(bid * VBLK, VBLK)
        row0 = pl.multiple_of(bid * rows_blk, rows_blk)
        pltpu.sync_copy(tt_hbm.at[:, pl.ds(v0, VBLK)], bt0)
        transpose(0)
        pltpu.sync_copy(bo0, lin_hbm.at[pl.ds(row0, rows_blk)])

    if tail:
      @pl.when(wid == nw - 1)
      def _():
        nr = tail * D // 128
        pltpu.sync_copy(tail_hbm, bo1.at[pl.ds(0, nr)])
        pltpu.sync_copy(bo1.at[pl.ds(0, nr)],
                        lin_hbm.at[pl.ds(n_full * rows_blk, nr)])

  return fmt_kernel


@functools.lru_cache(maxsize=None)
def _make_gather(S: int, Bt: int, D: int):
  info = plsc.get_sparse_core_info()
  nc, ns = info.num_cores, info.num_subcores
  nw = nc * ns
  assert Bt % (nw * 16) == 0 and D == EMBED_DIM
  assert CHUNK * nw == Bt
  n_tiles = CHUNK // 16
  mesh = plsc.VectorSubcoreMesh(core_axis_name="c", subcore_axis_name="s")

  half = CHUNK // 2
  n_half_tiles = half // 16
  assert S % 2 == 0

  @functools.partial(
      pl.kernel,
      mesh=mesh,
      out_type=jax.ShapeDtypeStruct((S, D, Bt), jnp.float32),
      compiler_params=pltpu.CompilerParams(needs_layout_passes=False),
      scratch_types=[
          pltpu.VMEM((CHUNK,), jnp.int32),
          pltpu.VMEM((CHUNK,), jnp.int32),
          pltpu.VMEM((CHUNK,), jnp.int32),
          pltpu.VMEM((half, 128), jnp.float32),
          pltpu.VMEM((half, 128), jnp.float32),
          pltpu.VMEM((D, CHUNK), jnp.float32),
          pltpu.VMEM((D, CHUNK), jnp.float32),
          pltpu.SemaphoreType.DMA,
          pltpu.SemaphoreType.DMA,
          pltpu.SemaphoreType.DMA,
          pltpu.SemaphoreType.DMA,
      ],
  )
  def gather_kernel(xt_hbm, t128_hbm, out_hbm, idx0, idx1, idxg,
                    rows0, rows1, tbuf0, tbuf1, isem, gsem0, gsem1, osem):
    wid = lax.axis_index("s") * nc + lax.axis_index("c")
    b0 = wid * CHUNK
    lane = lax.iota(jnp.int32, 16)
    rows = (rows0, rows1)
    gsems = (gsem0, gsem1)

    def idx_start(s, ib):
      pltpu.async_copy(xt_hbm.at[s, pl.ds(b0, CHUNK)], ib, isem)

    def idx_wait(s, ib):
      pltpu.make_async_copy(xt_hbm.at[s, pl.ds(b0, CHUNK)], ib, isem).wait()

    def shift(ib):
      @plsc.parallel_loop(0, n_tiles, 1, unroll=4)
      def _(t):
        idxg[pl.ds(t * 16, 16)] = (
            lax.shift_right_logical(ib[pl.ds(t * 16, 16)], 2))

    def g_start(j):
      pltpu.async_copy(t128_hbm.at[idxg.at[pl.ds(j * half, half)]],
                       rows[j], gsems[j])

    def g_wait(j):
      pltpu.make_async_copy(t128_hbm.at[idxg.at[pl.ds(j * half, half)]],
                            rows[j], gsems[j]).wait()

    def select(ib, j, tb):
      @plsc.parallel_loop(0, n_half_tiles, 1, unroll=4)
      def _(t):
        c0 = j * half + t * 16
        v = ib[pl.ds(c0, 16)]
        col0 = lax.shift_left(lax.bitwise_and(v, 3), 5)
        row = t * 16 + lane
        for d in range(D):
          tb[d, pl.ds(c0, 16)] = plsc.load_gather(rows[j], [row, col0 + d])

    def out_start(s, tb):
      pltpu.async_copy(tb, out_hbm.at[s, :, pl.ds(b0, CHUNK)], osem)

    def out_wait(s, tb):
      pltpu.make_async_copy(tb, out_hbm.at[s, :, pl.ds(b0, CHUNK)],
                            osem).wait()

    def step(s, ib_cur, ib_nxt, tb_cur, tb_prev):
      g_wait(0)

      @pl.when(s < S - 1)
      def _():
        idx_start(s + 1, ib_nxt)

      select(ib_cur, 0, tb_cur)
      g_wait(1)

      @pl.when(s < S - 1)
      def _():
        idx_wait(s + 1, ib_nxt)
        shift(ib_nxt)
        g_start(0)

      select(ib_cur, 1, tb_cur)

      @pl.when(s < S - 1)
      def _():
        g_start(1)

      @pl.when(s > 0)
      def _():
        out_wait(s - 1, tb_prev)

      out_start(s, tb_cur)

    pltpu.sync_copy(xt_hbm.at[0, pl.ds(b0, CHUNK)], idx0)
    shift(idx0)
    g_start(0)
    g_start(1)

    def body(s2, carry):
      step(2 * s2, idx0, idx1, tbuf0, tbuf1)
      step(2 * s2 + 1, idx1, idx0, tbuf1, tbuf0)
      return carry

    lax.fori_loop(0, S // 2, body, 0)
    out_wait(S - 1, tbuf1)

  return gather_kernel


def kernel(x, table):
  b, s = x.shape
  v, d = table.shape
  tt = table.T
  n_full = v // VBLK
  tail128 = table[n_full * VBLK:].reshape((v - n_full * VBLK) * d // 128, 128)
  lin = _make_reformat(v, d)(tt, tail128)
  xt = x.T
  out_t = _make_gather(s, b, d)(xt, lin)
  return out_t.transpose(2, 0, 1)


# speedup vs baseline: 1.0507x; 1.0507x over previous
import functools

import jax
import jax.numpy as jnp
from jax import lax
from jax.experimental import pallas as pl
from jax.experimental.pallas import tpu as pltpu
from jax.experimental.pallas import tpu_sc as plsc

EMBED_DIM = 32
CHUNK = 512
VBLK = 512


@functools.lru_cache(maxsize=None)
def _make_reformat(V: int, D: int):
  info = plsc.get_sparse_core_info()
  nc, ns = info.num_cores, info.num_subcores
  nw = nc * ns
  n_full = V // VBLK
  tail = V - n_full * VBLK
  per_w = n_full // nw
  n_extra = n_full - per_w * nw
  rows_blk = VBLK * D // 128
  mesh = plsc.VectorSubcoreMesh(core_axis_name="c", subcore_axis_name="s")
  assert per_w % 2 == 1 and per_w >= 3

  @functools.partial(
      pl.kernel,
      mesh=mesh,
      out_type=jax.ShapeDtypeStruct((V * D // 128, 128), jnp.float32),
      compiler_params=pltpu.CompilerParams(needs_layout_passes=False),
      scratch_types=[
          pltpu.VMEM((D, VBLK), jnp.float32),
          pltpu.VMEM((D, VBLK), jnp.float32),
          pltpu.VMEM((rows_blk, 128), jnp.float32),
          pltpu.VMEM((rows_blk, 128), jnp.float32),
          pltpu.SemaphoreType.DMA,
          pltpu.SemaphoreType.DMA,
          pltpu.SemaphoreType.DMA,
      ],
  )
  def fmt_kernel(tt_hbm, tail_hbm, lin_hbm, bt0, bt1, bo0, bo1,
                 isem, osem0, osem1):
    wid = lax.axis_index("s") * nc + lax.axis_index("c")
    lane = lax.iota(jnp.int32, 16)
    bts = (bt0, bt1)
    bos = (bo0, bo1)
    osems = (osem0, osem1)

    def bid_of(k):
      return wid * per_w + k

    def in_start(k, p):
      v0 = pl.multiple_of(bid_of(k) * VBLK, VBLK)
      pltpu.async_copy(tt_hbm.at[:, pl.ds(v0, VBLK)], bts[p], isem)

    def in_wait(k, p):
      v0 = pl.multiple_of(bid_of(k) * VBLK, VBLK)
      pltpu.make_async_copy(tt_hbm.at[:, pl.ds(v0, VBLK)], bts[p],
                            isem).wait()

    def transpose(p):
      @plsc.parallel_loop(0, VBLK // 16, 1, unroll=2)
      def _tp(t):
        rr = t * 16 + lane
        q = lax.shift_right_logical(rr, 2)
        r32 = lax.shift_left(lax.bitwise_and(rr, 3), 5)
        for d in range(D):
          plsc.store_scatter(bos[p], [q, r32 + d],
                             bts[p][d, pl.ds(t * 16, 16)])

    def out_start(k, p):
      row0 = pl.multiple_of(bid_of(k) * rows_blk, rows_blk)
      pltpu.async_copy(bos[p], lin_hbm.at[pl.ds(row0, rows_blk)], osems[p])

    def out_wait(k, p):
      row0 = pl.multiple_of(bid_of(k) * rows_blk, rows_blk)
      pltpu.make_async_copy(bos[p], lin_hbm.at[pl.ds(row0, rows_blk)],
                            osems[p]).wait()

    def step(k, p, last, first=False):
      in_wait(k, p)
      if not last:
        in_start(k + 1, 1 - p)
      if not first:
        out_wait(k - 2, p)
      transpose(p)
      out_start(k, p)

    in_start(0, 0)
    step(0, 0, False, first=True)
    step(1, 1, False, first=True)

    def body(k2, carry):
      k = 2 * k2
      step(k, 0, False)
      step(k + 1, 1, False)
      return carry

    lax.fori_loop(1, (per_w - 1) // 2, body, 0)
    step(per_w - 1, 0, True)
    out_wait(per_w - 2, 1)
    out_wait(per_w - 1, 0)

    if n_extra:
      @pl.when(wid < n_extra)
      def _():
        bid = n_full - n_extra + wid
        v0 = pl.multiple_of(bid * VBLK, VBLK)
        row0 = pl.multiple_of(bid * rows_blk, rows_blk)
        pltpu.sync_copy(tt_hbm.at[:, pl.ds(v0, VBLK)], bt0)
        transpose(0)
        pltpu.sync_copy(bo0, lin_hbm.at[pl.ds(row0, rows_blk)])

    if tail:
      @pl.when(wid == nw - 1)
      def _():
        nr = tail * D // 128
        pltpu.sync_copy(tail_hbm, bo1.at[pl.ds(0, nr)])
        pltpu.sync_copy(bo1.at[pl.ds(0, nr)],
                        lin_hbm.at[pl.ds(n_full * rows_blk, nr)])

  return fmt_kernel


@functools.lru_cache(maxsize=None)
def _make_gather(S: int, Bt: int, D: int):
  info = plsc.get_sparse_core_info()
  nc, ns = info.num_cores, info.num_subcores
  nw = nc * ns
  assert Bt % (nw * 16) == 0 and D == EMBED_DIM
  assert CHUNK * nw == Bt
  n_tiles = CHUNK // 16
  mesh = plsc.VectorSubcoreMesh(core_axis_name="c", subcore_axis_name="s")

  half = CHUNK // 2
  n_half_tiles = half // 16
  assert S % 2 == 0

  @functools.partial(
      pl.kernel,
      mesh=mesh,
      out_type=jax.ShapeDtypeStruct((S, D, Bt), jnp.float32),
      compiler_params=pltpu.CompilerParams(needs_layout_passes=False),
      scratch_types=[
          pltpu.VMEM((CHUNK,), jnp.int32),
          pltpu.VMEM((CHUNK,), jnp.int32),
          pltpu.VMEM((CHUNK,), jnp.int32),
          pltpu.VMEM((CHUNK,), jnp.int32),
          pltpu.VMEM((half, 128), jnp.float32),
          pltpu.VMEM((half, 128), jnp.float32),
          pltpu.VMEM((D, CHUNK), jnp.float32),
          pltpu.VMEM((D, CHUNK), jnp.float32),
          pltpu.SemaphoreType.DMA,
          pltpu.SemaphoreType.DMA,
          pltpu.SemaphoreType.DMA,
          pltpu.SemaphoreType.DMA,
      ],
  )
  def gather_kernel(xt_hbm, t128_hbm, out_hbm, idx0, idx1, idxg0, idxg1,
                    rows0, rows1, tbuf0, tbuf1, isem, gsem0, gsem1, osem):
    wid = lax.axis_index("s") * nc + lax.axis_index("c")
    b0 = wid * CHUNK
    lane = lax.iota(jnp.int32, 16)
    rows = (rows0, rows1)
    gsems = (gsem0, gsem1)

    def idx_start(s, ib):
      pltpu.async_copy(xt_hbm.at[s, pl.ds(b0, CHUNK)], ib, isem)

    def idx_wait(s, ib):
      pltpu.make_async_copy(xt_hbm.at[s, pl.ds(b0, CHUNK)], ib, isem).wait()

    def shift(ib, ig):
      @plsc.parallel_loop(0, n_tiles, 1, unroll=4)
      def _(t):
        ig[pl.ds(t * 16, 16)] = (
            lax.shift_right_logical(ib[pl.ds(t * 16, 16)], 2))

    def g_start(j, ig):
      pltpu.async_copy(t128_hbm.at[ig.at[pl.ds(j * half, half)]],
                       rows[j], gsems[j])

    def g_wait(j, ig):
      pltpu.make_async_copy(t128_hbm.at[ig.at[pl.ds(j * half, half)]],
                            rows[j], gsems[j]).wait()

    def select(ib, j, tb):
      @plsc.parallel_loop(0, n_half_tiles, 1, unroll=2)
      def _(t):
        c0 = j * half + t * 16
        v = ib[pl.ds(c0, 16)]
        col0 = lax.shift_left(lax.bitwise_and(v, 3), 5)
        row = t * 16 + lane
        for d in range(D):
          tb[d, pl.ds(c0, 16)] = plsc.load_gather(rows[j], [row, col0 + d])

    def out_start(s, tb):
      pltpu.async_copy(tb, out_hbm.at[s, :, pl.ds(b0, CHUNK)], osem)

    def out_wait(s, tb):
      pltpu.make_async_copy(tb, out_hbm.at[s, :, pl.ds(b0, CHUNK)],
                            osem).wait()

    def step(s, ib_cur, ib_nxt, ig_cur, ig_nxt, tb_cur, tb_prev):
      g_wait(0, ig_cur)

      @pl.when(s < S - 1)
      def _():
        idx_start(s + 1, ib_nxt)

      select(ib_cur, 0, tb_cur)

      @pl.when(s < S - 1)
      def _():
        idx_wait(s + 1, ib_nxt)
        shift(ib_nxt, ig_nxt)
        g_start(0, ig_nxt)

      g_wait(1, ig_cur)
      select(ib_cur, 1, tb_cur)

      @pl.when(s < S - 1)
      def _():
        g_start(1, ig_nxt)

      @pl.when(s > 0)
      def _():
        out_wait(s - 1, tb_prev)

      out_start(s, tb_cur)

    pltpu.sync_copy(xt_hbm.at[0, pl.ds(b0, CHUNK)], idx0)
    shift(idx0, idxg0)
    g_start(0, idxg0)
    g_start(1, idxg0)

    def body(s2, carry):
      step(2 * s2, idx0, idx1, idxg0, idxg1, tbuf0, tbuf1)
      step(2 * s2 + 1, idx1, idx0, idxg1, idxg0, tbuf1, tbuf0)
      return carry

    lax.fori_loop(0, S // 2, body, 0)
    out_wait(S - 1, tbuf1)

  return gather_kernel


def kernel(x, table):
  b, s = x.shape
  v, d = table.shape
  tt = table.T
  n_full = v // VBLK
  tail128 = table[n_full * VBLK:].reshape((v - n_full * VBLK) * d // 128, 128)
  lin = _make_reformat(v, d)(tt, tail128)
  xt = x.T
  out_t = _make_gather(s, b, d)(xt, lin)
  return out_t.transpose(2, 0, 1)
